# baseline (speedup 1.0000x reference)
import jax
import jax.numpy as jnp
from jax import lax
from jax.experimental import pallas as pl
from jax.experimental.pallas import tpu as pltpu
from jax.experimental.pallas import tpu_sc as plsc

_ROWS = 1_000_000
_D = 16

_NW = 32
_CH = 496
_NCHT = 1_008
_SC_ROWS = _NCHT * _CH
_MAXG = -(-_NCHT // _NW)

_TC_ROWS = _ROWS - _SC_ROWS
_TC_CH = 8_000
_TC_CHUNKS = [(_SC_ROWS + i * _TC_CH, _TC_CH) for i in range(_TC_ROWS // _TC_CH)]
_TC_CHUNKS.append((_SC_ROWS + (_TC_ROWS // _TC_CH) * _TC_CH, _TC_ROWS % _TC_CH))
_NBUF = 6
_LAG = 3


def _sc_run(src, dst, buf0, buf1, s_in0, s_in1, s_out0, s_out1):
    wid = lax.axis_index("s") * 2 + lax.axis_index("c")
    bufs = (buf0, buf1)
    sin = (s_in0, s_in1)
    sout = (s_out0, s_out1)

    def mk(g):
        cid = wid + g * _NW
        off = pl.multiple_of(cid * _CH, 8)
        b = g % 2
        inc = pltpu.make_async_copy(src.at[pl.ds(off, _CH), :], bufs[b], sin[b])
        outc = pltpu.make_async_copy(bufs[b], dst.at[pl.ds(off, _CH), :], sout[b])
        return cid, inc, outc

    cps = [mk(g) for g in range(_MAXG)]

    def when_valid(g, fn):
        pl.when(cps[g][0] < _NCHT)(fn)

    for g in range(_MAXG):
        if g >= 2:
            when_valid(g - 2, cps[g - 2][2].wait)
        when_valid(g, cps[g][1].start)
        if g >= 1:
            def drain(gg=g - 1):
                cps[gg][1].wait()
                cps[gg][2].start()
            when_valid(g - 1, drain)

    def tail(gg=_MAXG - 1):
        cps[gg][1].wait()
        cps[gg][2].start()
    when_valid(_MAXG - 1, tail)
    when_valid(_MAXG - 2, cps[_MAXG - 2][2].wait)
    when_valid(_MAXG - 1, cps[_MAXG - 1][2].wait)


def _tc_body(src, dst, *bufs_and_sems):
    bufs = bufs_and_sems[:_NBUF]
    sem_in, sem_out = bufs_and_sems[_NBUF], bufs_and_sems[_NBUF + 1]
    nsteps = len(_TC_CHUNKS)
    in_c = [None] * nsteps
    out_c = [None] * nsteps

    def issue_out(j):
        b = j % _NBUF
        off, sz = _TC_CHUNKS[j]
        in_c[j].wait()
        out_c[j] = pltpu.make_async_copy(
            bufs[b].at[pl.ds(0, sz), :],
            dst.at[pl.ds(off - _SC_ROWS, sz), :],
            sem_out.at[b],
        )
        out_c[j].start()

    for i in range(nsteps):
        b = i % _NBUF
        off, sz = _TC_CHUNKS[i]
        if i >= _NBUF:
            out_c[i - _NBUF].wait()
        in_c[i] = pltpu.make_async_copy(
            src.at[pl.ds(off, sz), :], bufs[b].at[pl.ds(0, sz), :], sem_in.at[b]
        )
        in_c[i].start()
        if i >= _LAG:
            issue_out(i - _LAG)
    for j in range(nsteps - _LAG, nsteps):
        issue_out(j)
    for j in range(nsteps - _NBUF, nsteps):
        out_c[j].wait()


def kernel(med_embeddings):
    sc_run = pl.kernel(
        _sc_run,
        out_type=jax.ShapeDtypeStruct((_ROWS, _D), jnp.float32),
        mesh=plsc.VectorSubcoreMesh(core_axis_name="c", subcore_axis_name="s"),
        scratch_types=[
            pltpu.VMEM((_CH, _D), jnp.float32),
            pltpu.VMEM((_CH, _D), jnp.float32),
            pltpu.SemaphoreType.DMA,
            pltpu.SemaphoreType.DMA,
            pltpu.SemaphoreType.DMA,
            pltpu.SemaphoreType.DMA,
        ],
    )
    lower = sc_run(med_embeddings)

    upper = pl.pallas_call(
        _tc_body,
        in_specs=[pl.BlockSpec(memory_space=pltpu.MemorySpace.HBM)],
        out_specs=pl.BlockSpec(memory_space=pltpu.MemorySpace.HBM),
        out_shape=jax.ShapeDtypeStruct((_TC_ROWS, _D), jnp.float32),
        scratch_shapes=(
            [pltpu.VMEM((_TC_CH, _D), jnp.float32) for _ in range(_NBUF)]
            + [pltpu.SemaphoreType.DMA((_NBUF,)), pltpu.SemaphoreType.DMA((_NBUF,))]
        ),
    )(med_embeddings)

    return lax.dynamic_update_slice(lower, upper, (_SC_ROWS, 0))


# speedup vs baseline: 1.0183x; 1.0183x over previous
import jax
import jax.numpy as jnp
from jax import lax
from jax.experimental import pallas as pl
from jax.experimental.pallas import tpu as pltpu
from jax.experimental.pallas import tpu_sc as plsc

_ROWS = 1_000_000
_D = 16
_NW = 32
_CH = 496
_NCHT = 2016
_PERW = _NCHT // _NW
_TAIL_OFF = _NCHT * _CH
_TAIL = _ROWS - _TAIL_OFF


def _run(src, dst, buf0, buf1, s_in0, s_in1, s_out0, s_out1):
    wid = lax.axis_index("s") * 2 + lax.axis_index("c")
    bufs = (buf0, buf1)
    sin = (s_in0, s_in1)
    sout = (s_out0, s_out1)

    def mk(g):
        cid = wid * _PERW + g
        off = pl.multiple_of(cid * _CH, 8)
        b = g % 2
        inc = pltpu.make_async_copy(src.at[pl.ds(off, _CH), :], bufs[b], sin[b])
        outc = pltpu.make_async_copy(bufs[b], dst.at[pl.ds(off, _CH), :], sout[b])
        return inc, outc

    cps = [mk(g) for g in range(_PERW)]
    for g in range(_PERW):
        if g >= 2:
            cps[g - 2][1].wait()
        cps[g][0].start()
        if g >= 1:
            cps[g - 1][0].wait()
            cps[g - 1][1].start()
    cps[_PERW - 1][0].wait()
    cps[_PERW - 1][1].start()
    cps[_PERW - 2][1].wait()
    cps[_PERW - 1][1].wait()

    tail_in = pltpu.make_async_copy(
        src.at[pl.ds(_TAIL_OFF, _TAIL), :], buf0.at[pl.ds(0, _TAIL), :], sin[0]
    )
    tail_out = pltpu.make_async_copy(
        buf0.at[pl.ds(0, _TAIL), :], dst.at[pl.ds(_TAIL_OFF, _TAIL), :], sout[0]
    )

    @pl.when(wid == 0)
    def _():
        tail_in.start()
        tail_in.wait()
        tail_out.start()
        tail_out.wait()


def kernel(med_embeddings):
    run = pl.kernel(
        _run,
        out_type=jax.ShapeDtypeStruct((_ROWS, _D), jnp.float32),
        mesh=plsc.VectorSubcoreMesh(core_axis_name="c", subcore_axis_name="s"),
        scratch_types=[
            pltpu.VMEM((_CH, _D), jnp.float32),
            pltpu.VMEM((_CH, _D), jnp.float32),
            pltpu.SemaphoreType.DMA,
            pltpu.SemaphoreType.DMA,
            pltpu.SemaphoreType.DMA,
            pltpu.SemaphoreType.DMA,
        ],
    )
    return run(med_embeddings)
